# baseline (speedup 1.0000x reference)
import functools

import jax
import jax.numpy as jnp
from jax import lax
from jax.experimental import pallas as pl
from jax.experimental.pallas import tpu as pltpu
from jax.experimental.pallas import tpu_sc as plsc

L = 16
NUM_WORKERS = 32
NBUF = 4
CH_ROWS = 16


def _row_maxc(buf, row, vecs):
    span = vecs // 4

    ms = []
    cs = []
    for k in range(4):
        ms.append(buf[row, pl.ds((k * span) * L, L)])
        cs.append(jnp.full((L,), k * span, jnp.int32))

    def vec_body(j, carry):
        out = []
        for k in range(4):
            m, c = carry[2 * k], carry[2 * k + 1]
            jj = k * span + j
            v = buf[row, pl.ds(jj * L, L)]
            gt = v > m
            m = jnp.where(gt, v, m)
            c = jnp.where(gt, jj, c)
            out += [m, c]
        return tuple(out)

    init = (ms[0], cs[0], ms[1], cs[1], ms[2], cs[2], ms[3], cs[3])
    carry = lax.fori_loop(1, span, vec_body, init, unroll=True)

    m, c = carry[0], carry[1]
    for k in range(1, 4):
        mk, ck = carry[2 * k], carry[2 * k + 1]
        gt = mk > m
        m = jnp.where(gt, mk, m)
        c = jnp.where(gt, ck, c)
    return m, c


@functools.lru_cache(maxsize=None)
def _build(rows, vocab):
    assert rows % (NUM_WORKERS * NBUF * CH_ROWS) == 0
    assert vocab % (4 * L) == 0
    rows_per_w = rows // NUM_WORKERS
    n_chunks = rows_per_w // CH_ROWS
    vecs = vocab // L

    mesh = plsc.VectorSubcoreMesh(core_axis_name="c", subcore_axis_name="s")

    @functools.partial(
        pl.kernel,
        out_type=jax.ShapeDtypeStruct((rows,), jnp.int32),
        mesh=mesh,
        compiler_params=pltpu.CompilerParams(needs_layout_passes=False),
        scratch_types=(
            [pltpu.VMEM((CH_ROWS, vocab), jnp.float32) for _ in range(NBUF)]
            + [pltpu.VMEM((rows_per_w,), jnp.int32),
               pltpu.VMEM((CH_ROWS * L,), jnp.float32),
               pltpu.VMEM((CH_ROWS * L,), jnp.int32)]
            + [pltpu.SemaphoreType.DMA for _ in range(NBUF)]
        ),
    )
    def k(x_hbm, out_hbm, b0, b1, b2, b3, out_v, mbuf, cbuf, s0, s1, s2, s3):
        bufs = (b0, b1, b2, b3)
        sems = (s0, s1, s2, s3)
        wid = lax.axis_index("s") * 2 + lax.axis_index("c")
        row0 = wid * rows_per_w

        def start(g, b):
            pltpu.async_copy(
                x_hbm.at[pl.ds(row0 + g * CH_ROWS, CH_ROWS)], bufs[b], sems[b])

        def wait(b):
            pltpu.make_async_copy(
                x_hbm.at[pl.ds(0, CH_ROWS)], bufs[b], sems[b]).wait()

        def process(g, b):
            buf = bufs[b]

            @plsc.parallel_loop(0, CH_ROWS, unroll=4)
            def row_body(r):
                m, c = _row_maxc(buf, r, vecs)
                mbuf[pl.ds(r * L, L)] = m
                cbuf[pl.ds(r * L, L)] = c

            lane = lax.iota(jnp.int32, L)
            stride = lane * L
            best = plsc.load_gather(mbuf, [stride])
            besti = plsc.load_gather(cbuf, [stride]) * L
            for l in range(1, L):
                v = plsc.load_gather(mbuf, [stride + l])
                vi = plsc.load_gather(cbuf, [stride + l]) * L + l
                gt = v > best
                eq = v == best
                lt = vi < besti
                upd = gt | (eq & lt)
                best = jnp.where(upd, v, best)
                besti = jnp.where(upd, vi, besti)
            out_v[pl.ds(g * CH_ROWS, L)] = besti

        for b in range(NBUF):
            start(b, b)

        def ring(i, _):
            for b in range(NBUF):
                g = i * NBUF + b
                wait(b)
                process(g, b)
                start(g + NBUF, b)
            return 0

        lax.fori_loop(0, n_chunks // NBUF - 1, ring, 0)
        for b in range(NBUF):
            g = n_chunks - NBUF + b
            wait(b)
            process(g, b)

        pltpu.sync_copy(out_v, out_hbm.at[pl.ds(wid * rows_per_w, rows_per_w)])

    return k


def kernel(log_probs):
    b, t, v = log_probs.shape
    out = _build(b * t, v)(log_probs.reshape(b * t, v))
    return out.reshape(b, t)


# speedup vs baseline: 1.0603x; 1.0603x over previous
import functools

import jax
import jax.numpy as jnp
from jax import lax
from jax.experimental import pallas as pl
from jax.experimental.pallas import tpu as pltpu
from jax.experimental.pallas import tpu_sc as plsc

L = 16
NUM_WORKERS = 32
NBUF = 4
CH_ROWS = 16


def _row_maxc(buf, row, vecs):
    span = vecs // 4

    ms = []
    cs = []
    for k in range(4):
        ms.append(buf[row, pl.ds((k * span) * L, L)])
        cs.append(jnp.full((L,), k * span, jnp.int32))

    def vec_body(j, carry):
        out = []
        for k in range(4):
            m, c = carry[2 * k], carry[2 * k + 1]
            jj = k * span + j
            v = buf[row, pl.ds(jj * L, L)]
            gt = v > m
            m = jnp.where(gt, v, m)
            c = jnp.where(gt, jj, c)
            out += [m, c]
        return tuple(out)

    init = (ms[0], cs[0], ms[1], cs[1], ms[2], cs[2], ms[3], cs[3])
    carry = lax.fori_loop(1, span, vec_body, init, unroll=True)

    m, c = carry[0], carry[1]
    for k in range(1, 4):
        mk, ck = carry[2 * k], carry[2 * k + 1]
        gt = mk > m
        m = jnp.where(gt, mk, m)
        c = jnp.where(gt, ck, c)
    return m, c


@functools.lru_cache(maxsize=None)
def _build(rows, vocab):
    assert rows % (NUM_WORKERS * NBUF * CH_ROWS) == 0
    assert vocab % (4 * L) == 0
    rows_per_w = rows // NUM_WORKERS
    n_chunks = rows_per_w // CH_ROWS
    vecs = vocab // L

    mesh = plsc.VectorSubcoreMesh(core_axis_name="c", subcore_axis_name="s")

    @functools.partial(
        pl.kernel,
        out_type=jax.ShapeDtypeStruct((rows,), jnp.int32),
        mesh=mesh,
        compiler_params=pltpu.CompilerParams(needs_layout_passes=False),
        scratch_types=(
            [pltpu.VMEM((CH_ROWS, vocab), jnp.float32) for _ in range(NBUF)]
            + [pltpu.VMEM((rows_per_w,), jnp.int32),
               pltpu.VMEM((2 * CH_ROWS * L,), jnp.float32),
               pltpu.VMEM((2 * CH_ROWS * L,), jnp.int32)]
            + [pltpu.SemaphoreType.DMA for _ in range(NBUF)]
        ),
    )
    def k(x_hbm, out_hbm, b0, b1, b2, b3, out_v, mbuf, cbuf, s0, s1, s2, s3):
        bufs = (b0, b1, b2, b3)
        sems = (s0, s1, s2, s3)
        wid = lax.axis_index("s") * 2 + lax.axis_index("c")
        row0 = wid * rows_per_w

        def start(g, b):
            pltpu.async_copy(
                x_hbm.at[pl.ds(row0 + g * CH_ROWS, CH_ROWS)], bufs[b], sems[b])

        def wait(b):
            pltpu.make_async_copy(
                x_hbm.at[pl.ds(0, CH_ROWS)], bufs[b], sems[b]).wait()

        def process(g, b, par):
            buf = bufs[b]
            off = par * CH_ROWS * L

            @plsc.parallel_loop(0, CH_ROWS, unroll=2)
            def row_body(r):
                m, c = _row_maxc(buf, r, vecs)
                mbuf[pl.ds(off + r * L, L)] = m
                cbuf[pl.ds(off + r * L, L)] = c

            lane = lax.iota(jnp.int32, L)
            stride = lane * L + off
            pairs = []
            for l in range(L):
                v = plsc.load_gather(mbuf, [stride + l])
                vi = plsc.load_gather(cbuf, [stride + l]) * L + l
                pairs.append((v, vi))
            while len(pairs) > 1:
                nxt = []
                for a in range(0, len(pairs), 2):
                    (v1, i1), (v2, i2) = pairs[a], pairs[a + 1]
                    upd = (v2 > v1) | ((v2 == v1) & (i2 < i1))
                    nxt.append((jnp.where(upd, v2, v1),
                                jnp.where(upd, i2, i1)))
                pairs = nxt
            out_v[pl.ds(g * CH_ROWS, L)] = pairs[0][1]

        for b in range(NBUF):
            start(b, b)

        def ring(i, _):
            for b in range(NBUF):
                g = i * NBUF + b
                wait(b)
                process(g, b, b % 2)
                start(g + NBUF, b)
            return 0

        lax.fori_loop(0, n_chunks // NBUF - 1, ring, 0)
        for b in range(NBUF):
            g = n_chunks - NBUF + b
            wait(b)
            process(g, b, b % 2)

        pltpu.sync_copy(out_v, out_hbm.at[pl.ds(wid * rows_per_w, rows_per_w)])

    return k


def kernel(log_probs):
    b, t, v = log_probs.shape
    out = _build(b * t, v)(log_probs.reshape(b * t, v))
    return out.reshape(b, t)


# speedup vs baseline: 1.0938x; 1.0317x over previous
import functools

import jax
import jax.numpy as jnp
from jax import lax
from jax.experimental import pallas as pl
from jax.experimental.pallas import tpu as pltpu
from jax.experimental.pallas import tpu_sc as plsc

L = 16
NUM_WORKERS = 32
NBUF = 4
CH_ROWS = 16


def _row_maxc(buf, row, vecs):
    span = vecs // 4

    ms = []
    cs = []
    for k in range(4):
        ms.append(buf[row, pl.ds((k * span) * L, L)])
        cs.append(jnp.full((L,), k * span, jnp.int32))

    def vec_body(j, carry):
        out = []
        for k in range(4):
            m, c = carry[2 * k], carry[2 * k + 1]
            jj = k * span + j
            v = buf[row, pl.ds(jj * L, L)]
            gt = v > m
            m = jnp.where(gt, v, m)
            c = jnp.where(gt, jj, c)
            out += [m, c]
        return tuple(out)

    init = (ms[0], cs[0], ms[1], cs[1], ms[2], cs[2], ms[3], cs[3])
    carry = lax.fori_loop(1, span, vec_body, init, unroll=True)

    m, c = carry[0], carry[1]
    for k in range(1, 4):
        mk, ck = carry[2 * k], carry[2 * k + 1]
        gt = mk > m
        m = jnp.where(gt, mk, m)
        c = jnp.where(gt, ck, c)
    return m, c


@functools.lru_cache(maxsize=None)
def _build(total_rows, sc_rows, vocab):
    assert sc_rows % (NUM_WORKERS * NBUF * CH_ROWS) == 0
    assert vocab % (4 * L) == 0
    offset = total_rows - sc_rows
    rows_per_w = sc_rows // NUM_WORKERS
    n_chunks = rows_per_w // CH_ROWS
    vecs = vocab // L

    mesh = plsc.VectorSubcoreMesh(core_axis_name="c", subcore_axis_name="s")

    @functools.partial(
        pl.kernel,
        out_type=jax.ShapeDtypeStruct((sc_rows,), jnp.int32),
        mesh=mesh,
        compiler_params=pltpu.CompilerParams(needs_layout_passes=False),
        scratch_types=(
            [pltpu.VMEM((CH_ROWS, vocab), jnp.float32) for _ in range(NBUF)]
            + [pltpu.VMEM((rows_per_w,), jnp.int32),
               pltpu.VMEM((2 * CH_ROWS * L,), jnp.float32),
               pltpu.VMEM((2 * CH_ROWS * L,), jnp.int32)]
            + [pltpu.SemaphoreType.DMA for _ in range(NBUF)]
        ),
    )
    def k(x_hbm, out_hbm, b0, b1, b2, b3, out_v, mbuf, cbuf, s0, s1, s2, s3):
        bufs = (b0, b1, b2, b3)
        sems = (s0, s1, s2, s3)
        wid = lax.axis_index("s") * 2 + lax.axis_index("c")
        row0 = offset + wid * rows_per_w

        def start(g, b):
            pltpu.async_copy(
                x_hbm.at[pl.ds(row0 + g * CH_ROWS, CH_ROWS)], bufs[b], sems[b])

        def wait(b):
            pltpu.make_async_copy(
                x_hbm.at[pl.ds(0, CH_ROWS)], bufs[b], sems[b]).wait()

        def process(g, b, par):
            buf = bufs[b]
            off = par * CH_ROWS * L

            @plsc.parallel_loop(0, CH_ROWS, unroll=2)
            def row_body(r):
                m, c = _row_maxc(buf, r, vecs)
                mbuf[pl.ds(off + r * L, L)] = m
                cbuf[pl.ds(off + r * L, L)] = c

            lane = lax.iota(jnp.int32, L)
            stride = lane * L + off
            pairs = []
            for l in range(L):
                v = plsc.load_gather(mbuf, [stride + l])
                vi = plsc.load_gather(cbuf, [stride + l]) * L + l
                pairs.append((v, vi))
            while len(pairs) > 1:
                nxt = []
                for a in range(0, len(pairs), 2):
                    (v1, i1), (v2, i2) = pairs[a], pairs[a + 1]
                    upd = (v2 > v1) | ((v2 == v1) & (i2 < i1))
                    nxt.append((jnp.where(upd, v2, v1),
                                jnp.where(upd, i2, i1)))
                pairs = nxt
            out_v[pl.ds(g * CH_ROWS, L)] = pairs[0][1]

        for b in range(NBUF):
            start(b, b)

        def ring(i, _):
            for b in range(NBUF):
                g = i * NBUF + b
                wait(b)
                process(g, b, b % 2)
                start(g + NBUF, b)
            return 0

        lax.fori_loop(0, n_chunks // NBUF - 1, ring, 0)
        for b in range(NBUF):
            g = n_chunks - NBUF + b
            wait(b)
            process(g, b, b % 2)

        pltpu.sync_copy(out_v, out_hbm.at[pl.ds(wid * rows_per_w, rows_per_w)])

    return k


TC_FRAC_NUM, TC_FRAC_DEN = 1, 2
TC_BLOCK_ROWS = 512


@functools.lru_cache(maxsize=None)
def _build_tc(tc_rows, vocab):
    nb = tc_rows // TC_BLOCK_ROWS

    def body(x_ref, o_ref):
        xb = x_ref[...]
        m = jnp.max(xb, axis=1, keepdims=True)
        iota = lax.broadcasted_iota(jnp.int32, xb.shape, 1)
        cand = jnp.where(xb == m, iota, vocab)
        o_ref[0, 0, :] = jnp.min(cand, axis=1)

    return pl.pallas_call(
        body,
        grid=(nb,),
        in_specs=[pl.BlockSpec((TC_BLOCK_ROWS, vocab), lambda i: (i, 0))],
        out_specs=pl.BlockSpec((1, 1, TC_BLOCK_ROWS), lambda i: (i, 0, 0)),
        out_shape=jax.ShapeDtypeStruct((nb, 1, TC_BLOCK_ROWS), jnp.int32),
    )


def kernel(log_probs):
    b, t, v = log_probs.shape
    rows = b * t
    x2 = log_probs.reshape(rows, v)
    sc_quantum = NUM_WORKERS * NBUF * CH_ROWS
    tc_rows = (rows * TC_FRAC_NUM // TC_FRAC_DEN) // sc_quantum * sc_quantum
    sc_out = _build(rows, rows - tc_rows, v)(x2)
    if tc_rows:
        tc_out = _build_tc(tc_rows, v)(x2).reshape(tc_rows)
        out = jnp.concatenate([tc_out, sc_out])
    else:
        out = sc_out
    return out.reshape(b, t)


# speedup vs baseline: 1.1572x; 1.0580x over previous
import functools

import jax
import jax.numpy as jnp
from jax import lax
from jax.experimental import pallas as pl
from jax.experimental.pallas import tpu as pltpu
from jax.experimental.pallas import tpu_sc as plsc

L = 16
NUM_WORKERS = 32
NBUF = 4
CH_ROWS = 16


def _row_maxc(buf, row, vecs):
    span = vecs // 4

    ms = []
    cs = []
    for k in range(4):
        ms.append(buf[row, pl.ds((k * span) * L, L)])
        cs.append(jnp.full((L,), k * span, jnp.int32))

    def vec_body(j, carry):
        out = []
        for k in range(4):
            m, c = carry[2 * k], carry[2 * k + 1]
            jj = k * span + j
            v = buf[row, pl.ds(jj * L, L)]
            gt = v > m
            m = jnp.where(gt, v, m)
            c = jnp.where(gt, jj, c)
            out += [m, c]
        return tuple(out)

    init = (ms[0], cs[0], ms[1], cs[1], ms[2], cs[2], ms[3], cs[3])
    carry = lax.fori_loop(1, span, vec_body, init, unroll=True)

    m, c = carry[0], carry[1]
    for k in range(1, 4):
        mk, ck = carry[2 * k], carry[2 * k + 1]
        gt = mk > m
        m = jnp.where(gt, mk, m)
        c = jnp.where(gt, ck, c)
    return m, c


@functools.lru_cache(maxsize=None)
def _build(total_rows, sc_rows, vocab):
    assert sc_rows % (NUM_WORKERS * NBUF * CH_ROWS) == 0
    assert vocab % (4 * L) == 0
    offset = total_rows - sc_rows
    rows_per_w = sc_rows // NUM_WORKERS
    n_chunks = rows_per_w // CH_ROWS
    vecs = vocab // L

    mesh = plsc.VectorSubcoreMesh(core_axis_name="c", subcore_axis_name="s")

    @functools.partial(
        pl.kernel,
        out_type=jax.ShapeDtypeStruct((sc_rows,), jnp.int32),
        mesh=mesh,
        compiler_params=pltpu.CompilerParams(needs_layout_passes=False),
        scratch_types=(
            [pltpu.VMEM((CH_ROWS, vocab), jnp.float32) for _ in range(NBUF)]
            + [pltpu.VMEM((rows_per_w,), jnp.int32),
               pltpu.VMEM((2 * CH_ROWS * L,), jnp.float32),
               pltpu.VMEM((2 * CH_ROWS * L,), jnp.int32)]
            + [pltpu.SemaphoreType.DMA for _ in range(NBUF)]
        ),
    )
    def k(x_hbm, out_hbm, b0, b1, b2, b3, out_v, mbuf, cbuf, s0, s1, s2, s3):
        bufs = (b0, b1, b2, b3)
        sems = (s0, s1, s2, s3)
        wid = lax.axis_index("s") * 2 + lax.axis_index("c")
        row0 = offset + wid * rows_per_w

        def start(g, b):
            pltpu.async_copy(
                x_hbm.at[pl.ds(row0 + g * CH_ROWS, CH_ROWS)], bufs[b], sems[b])

        def wait(b):
            pltpu.make_async_copy(
                x_hbm.at[pl.ds(0, CH_ROWS)], bufs[b], sems[b]).wait()

        def process(g, b, par):
            buf = bufs[b]
            off = par * CH_ROWS * L

            @plsc.parallel_loop(0, CH_ROWS, unroll=2)
            def row_body(r):
                m, c = _row_maxc(buf, r, vecs)
                mbuf[pl.ds(off + r * L, L)] = m
                cbuf[pl.ds(off + r * L, L)] = c

            lane = lax.iota(jnp.int32, L)
            stride = lane * L + off
            pairs = []
            for l in range(L):
                v = plsc.load_gather(mbuf, [stride + l])
                vi = plsc.load_gather(cbuf, [stride + l]) * L + l
                pairs.append((v, vi))
            while len(pairs) > 1:
                nxt = []
                for a in range(0, len(pairs), 2):
                    (v1, i1), (v2, i2) = pairs[a], pairs[a + 1]
                    upd = (v2 > v1) | ((v2 == v1) & (i2 < i1))
                    nxt.append((jnp.where(upd, v2, v1),
                                jnp.where(upd, i2, i1)))
                pairs = nxt
            out_v[pl.ds(g * CH_ROWS, L)] = pairs[0][1]

        for b in range(NBUF):
            start(b, b)

        def ring(i, _):
            for b in range(NBUF):
                g = i * NBUF + b
                wait(b)
                process(g, b, b % 2)
                start(g + NBUF, b)
            return 0

        lax.fori_loop(0, n_chunks // NBUF - 1, ring, 0)
        for b in range(NBUF):
            g = n_chunks - NBUF + b
            wait(b)
            process(g, b, b % 2)

        pltpu.sync_copy(out_v, out_hbm.at[pl.ds(wid * rows_per_w, rows_per_w)])

    return k


TC_FRAC_NUM, TC_FRAC_DEN = 7, 16
TC_BLOCK_ROWS = 512


@functools.lru_cache(maxsize=None)
def _build_tc(tc_rows, vocab):
    nb = tc_rows // TC_BLOCK_ROWS

    def body(x_ref, o_ref):
        xb = x_ref[...]
        m = jnp.max(xb, axis=1, keepdims=True)
        iota = lax.broadcasted_iota(jnp.int32, xb.shape, 1)
        cand = jnp.where(xb == m, iota, vocab)
        o_ref[0, 0, :] = jnp.min(cand, axis=1)

    return pl.pallas_call(
        body,
        grid=(nb,),
        in_specs=[pl.BlockSpec((TC_BLOCK_ROWS, vocab), lambda i: (i, 0))],
        out_specs=pl.BlockSpec((1, 1, TC_BLOCK_ROWS), lambda i: (i, 0, 0)),
        out_shape=jax.ShapeDtypeStruct((nb, 1, TC_BLOCK_ROWS), jnp.int32),
    )


def kernel(log_probs):
    b, t, v = log_probs.shape
    rows = b * t
    x2 = log_probs.reshape(rows, v)
    sc_quantum = NUM_WORKERS * NBUF * CH_ROWS
    tc_rows = (rows * TC_FRAC_NUM // TC_FRAC_DEN) // sc_quantum * sc_quantum
    sc_out = _build(rows, rows - tc_rows, v)(x2)
    if tc_rows:
        tc_out = _build_tc(tc_rows, v)(x2).reshape(tc_rows)
        out = jnp.concatenate([tc_out, sc_out])
    else:
        out = sc_out
    return out.reshape(b, t)
